# baseline (speedup 1.0000x reference)
import functools

import jax
import jax.numpy as jnp
from jax import lax
from jax.experimental import pallas as pl
from jax.experimental.pallas import tpu as pltpu
from jax.experimental.pallas import tpu_sc as plsc

NC = 2
NS = 16
NW = NC * NS
LANES = 16

BATCH = 16384
D_CTX = 64
B_PER_W = BATCH // NW
CHUNK = 128
NCHUNK = B_PER_W // CHUNK


def _emb_body(ctx_hbm, lab_hbm, tab_hbm, out_hbm, idx_v, acc_v, rows_v,
              sem_ctx, sem_a, sem_b):
    wid = lax.axis_index("s") * NC + lax.axis_index("c")
    base = wid * B_PER_W

    ctx_cp = pltpu.async_copy(ctx_hbm.at[pl.ds(base, B_PER_W)], acc_v, sem_ctx)
    pltpu.sync_copy(lab_hbm.at[wid], idx_v)

    sems = (sem_a, sem_b)
    gathers = [
        pltpu.async_copy(tab_hbm.at[idx_v.at[0]], rows_v.at[0], sems[0])
    ]
    ctx_cp.wait()

    for c in range(NCHUNK):
        if c + 1 < NCHUNK:
            gathers.append(
                pltpu.async_copy(
                    tab_hbm.at[idx_v.at[c + 1]],
                    rows_v.at[(c + 1) % 2],
                    sems[(c + 1) % 2],
                )
            )
        gathers[c].wait()
        buf = c % 2

        def add_row(r, carry, _c=c, _buf=buf):
            for j in range(D_CTX // LANES):
                sl = pl.ds(j * LANES, LANES)
                row = _c * CHUNK + r
                acc_v[row, sl] = acc_v[row, sl] + rows_v[_buf, r, sl]
            return carry

        lax.fori_loop(0, CHUNK, add_row, 0, unroll=4)

    pltpu.sync_copy(acc_v, out_hbm.at[pl.ds(base, B_PER_W)])


@functools.partial(jax.jit, static_argnames=())
def _emb_call(ctx_vec, labels_blocked, emb_weight):
    mesh = plsc.VectorSubcoreMesh(
        core_axis_name="c", subcore_axis_name="s", num_cores=NC, num_subcores=NS
    )
    run = pl.kernel(
        _emb_body,
        out_type=jax.ShapeDtypeStruct((BATCH, D_CTX), jnp.float32),
        mesh=mesh,
        scratch_types=[
            pltpu.VMEM((NCHUNK, CHUNK), jnp.int32),
            pltpu.VMEM((B_PER_W, D_CTX), jnp.float32),
            pltpu.VMEM((2, CHUNK, D_CTX), jnp.float32),
            pltpu.SemaphoreType.DMA,
            pltpu.SemaphoreType.DMA,
            pltpu.SemaphoreType.DMA,
        ],
        compiler_params=pltpu.CompilerParams(use_tc_tiling_on_sc=False),
    )
    return run(ctx_vec, labels_blocked, emb_weight)


def kernel(ctx_vec, labels, emb_weight):
    labels_blocked = jnp.reshape(labels.astype(jnp.int32), (NW, NCHUNK, CHUNK))
    return _emb_call(ctx_vec, labels_blocked, emb_weight)


# speedup vs baseline: 1.8606x; 1.8606x over previous
import jax
import jax.numpy as jnp
from jax import lax
from jax.experimental import pallas as pl
from jax.experimental.pallas import tpu as pltpu
from jax.experimental.pallas import tpu_sc as plsc

NC = 2
NS = 16
NW = NC * NS
LANES = 16

BATCH = 16384
D_CTX = 64
VOCAB = 1000000
B_PER_W = BATCH // NW
NBLK = VOCAB // 128 + 1
NVEC = BATCH // LANES
NRING = 4
NCOL = 16


def _iota16():
    return lax.iota(jnp.int32, LANES)


def _splat(x):
    return jnp.full((LANES,), x, jnp.int32)


def _gather_body(lab_hbm, tab_hbm, tail_hbm, g_hbm, dump_hbm,
                 labv, cnt, sta, cur, dlab, dpos,
                 r0, r1, r2, r3, ct, *sems):
    sblk = sems[:NRING]
    scol = sems[NRING:NRING + NCOL]
    rings = (r0, r1, r2, r3)
    wid = lax.axis_index("s") * NC + lax.axis_index("c")
    b0 = wid * NBLK // NW
    b1 = (wid + 1) * NBLK // NW
    nb = b1 - b0
    iota = _iota16()

    pltpu.sync_copy(lab_hbm, labv)
    for i in range(256 // LANES):
        z = jnp.zeros((LANES,), jnp.int32)
        cnt[pl.ds(i * LANES, LANES)] = z

    for k in range(NCOL):
        pltpu.make_async_copy(
            ct.at[k], dump_hbm.at[pl.ds((wid * NCOL + k) * D_CTX, D_CTX)],
            scol[k],
        ).start()

    def count_pass(v, carry):
        lab = labv[pl.ds(v * LANES, LANES)]
        blk = lax.shift_right_logical(lab, 7)
        m = (blk >= b0) & (blk < b1)
        bl = jnp.clip(blk - b0, 0, 255)
        rank, last = plsc.scan_count(bl, m)
        old = plsc.load_gather(cnt, [bl], mask=m)
        plsc.store_scatter(cnt, [bl], old + rank + 1, mask=m & last)
        return carry

    lax.fori_loop(0, NVEC, count_pass, 0)

    carry = jnp.int32(0)
    for i in range(256 // LANES):
        c16 = cnt[pl.ds(i * LANES, LANES)]
        cs = plsc.cumsum(c16)
        sta[pl.ds(i * LANES, LANES)] = cs - c16 + carry
        carry = carry + jnp.sum(c16)
    for i in range(256 // LANES):
        cur[pl.ds(i * LANES, LANES)] = sta[pl.ds(i * LANES, LANES)]

    def place_pass(v, carry):
        lab = labv[pl.ds(v * LANES, LANES)]
        blk = lax.shift_right_logical(lab, 7)
        m = (blk >= b0) & (blk < b1)
        bl = jnp.clip(blk - b0, 0, 255)
        rank, last = plsc.scan_count(bl, m)
        c = plsc.load_gather(cur, [bl], mask=m)
        slot = jnp.clip(c + rank, 0, BATCH - 1)
        plsc.store_scatter(dlab, [slot], lab, mask=m)
        plsc.store_scatter(dpos, [slot], v * LANES + iota, mask=m)
        plsc.store_scatter(cur, [bl], c + rank + 1, mask=m & last)
        return carry

    lax.fori_loop(0, NVEC, place_pass, 0)

    def fire_block(b, k):
        blk_g = b0 + b

        @pl.when(blk_g < NBLK - 1)
        def _():
            off = pl.multiple_of(blk_g * 128, 128)
            pltpu.make_async_copy(
                tab_hbm.at[:, pl.ds(off, 128)], rings[k], sblk[k]
            ).start()

        @pl.when(blk_g >= NBLK - 1)
        def _():
            pltpu.make_async_copy(tail_hbm, rings[k], sblk[k]).start()

    for k in range(NRING):
        @pl.when(k < nb)
        def _(k=k):
            fire_block(k, k)

    def process_group(g, carry):
        for k in range(NRING):
            b = g * NRING + k

            @pl.when(b < nb)
            def _(b=b, k=k):
                pltpu.make_async_copy(
                    tab_hbm.at[:, pl.ds(0, 128)], rings[k], sblk[k]
                ).wait()
                n = jnp.max(plsc.load_gather(cnt, [_splat(b)]))
                start = jnp.max(plsc.load_gather(sta, [_splat(b)]))

                @pl.when(n > 0)
                def _():
                    def hit_group(hg, carry2):
                        for kc in range(NCOL):
                            h = hg * NCOL + kc

                            @pl.when(h < n)
                            def _(h=h, kc=kc):
                                j = start + h
                                labs = plsc.load_gather(dlab, [_splat(j)])
                                poss = plsc.load_gather(dpos, [_splat(j)])
                                lane = labs & 127
                                pltpu.make_async_copy(
                                    ct.at[kc],
                                    dump_hbm.at[pl.ds(0, D_CTX)],
                                    scol[kc],
                                ).wait()
                                for jj in range(D_CTX // LANES):
                                    vals = plsc.load_gather(
                                        rings[k], [iota + jj * LANES, lane]
                                    )
                                    ct[kc, pl.ds(jj * LANES, LANES)] = vals
                                p = jnp.max(poss)
                                pltpu.make_async_copy(
                                    ct.at[kc],
                                    g_hbm.at[pl.ds(p * D_CTX, D_CTX)],
                                    scol[kc],
                                ).start()

                        return carry2

                    lax.fori_loop(0, (n + NCOL - 1) // NCOL, hit_group, 0)

                @pl.when(b + NRING < nb)
                def _(b=b, k=k):
                    fire_block(b + NRING, k)

        return carry

    lax.fori_loop(0, (nb + NRING - 1) // NRING, process_group, 0)

    for k in range(NCOL):
        pltpu.make_async_copy(
            ct.at[k], dump_hbm.at[pl.ds(0, D_CTX)], scol[k]
        ).wait()


def _add_body(ctx_hbm, g_hbm, out_hbm, gv, cv, ov, sem_g, sem_c):
    wid = lax.axis_index("s") * NC + lax.axis_index("c")
    base = wid * B_PER_W
    iota = _iota16()
    cp_g = pltpu.async_copy(
        g_hbm.at[pl.ds(base * D_CTX, B_PER_W * D_CTX)], gv, sem_g
    )
    cp_c = pltpu.async_copy(ctx_hbm.at[:, pl.ds(base, B_PER_W)], cv, sem_c)
    cp_g.wait()
    cp_c.wait()

    def row(j, carry):
        for pg in range(B_PER_W // LANES):
            idx = (iota + pg * LANES) * D_CTX + j
            vals = plsc.load_gather(gv, [idx])
            sl = pl.ds(pg * LANES, LANES)
            ov[j, sl] = vals + cv[j, sl]
        return carry

    lax.fori_loop(0, D_CTX, row, 0)
    pltpu.sync_copy(ov, out_hbm.at[:, pl.ds(base, B_PER_W)])


@jax.jit
def _emb_call(ctx_t, labels, tab_t, tab_tail):
    mesh = plsc.VectorSubcoreMesh(
        core_axis_name="c", subcore_axis_name="s", num_cores=NC, num_subcores=NS
    )
    gather = pl.kernel(
        _gather_body,
        out_type=(
            jax.ShapeDtypeStruct((BATCH * D_CTX,), jnp.float32),
            jax.ShapeDtypeStruct((NW * NCOL * D_CTX,), jnp.float32),
        ),
        mesh=mesh,
        scratch_types=[
            pltpu.VMEM((BATCH,), jnp.int32),
            pltpu.VMEM((256,), jnp.int32),
            pltpu.VMEM((256,), jnp.int32),
            pltpu.VMEM((256,), jnp.int32),
            pltpu.VMEM((BATCH,), jnp.int32),
            pltpu.VMEM((BATCH,), jnp.int32),
            pltpu.VMEM((D_CTX, 128), jnp.float32),
            pltpu.VMEM((D_CTX, 128), jnp.float32),
            pltpu.VMEM((D_CTX, 128), jnp.float32),
            pltpu.VMEM((D_CTX, 128), jnp.float32),
            pltpu.VMEM((NCOL, D_CTX), jnp.float32),
        ] + [pltpu.SemaphoreType.DMA] * (NRING + NCOL),
        compiler_params=pltpu.CompilerParams(needs_layout_passes=False),
    )
    g, _ = gather(labels, tab_t, tab_tail)
    add = pl.kernel(
        _add_body,
        out_type=jax.ShapeDtypeStruct((D_CTX, BATCH), jnp.float32),
        mesh=mesh,
        scratch_types=[
            pltpu.VMEM((B_PER_W * D_CTX,), jnp.float32),
            pltpu.VMEM((D_CTX, B_PER_W), jnp.float32),
            pltpu.VMEM((D_CTX, B_PER_W), jnp.float32),
            pltpu.SemaphoreType.DMA,
            pltpu.SemaphoreType.DMA,
        ],
        compiler_params=pltpu.CompilerParams(needs_layout_passes=False),
    )
    return add(ctx_t, g)


def kernel(ctx_vec, labels, emb_weight):
    tail = jnp.pad(emb_weight[(NBLK - 1) * 128:], ((0, 64), (0, 0))).T
    out_t = _emb_call(
        ctx_vec.T, labels.astype(jnp.int32), emb_weight.T, tail
    )
    return out_t.T
